# baseline (speedup 1.0000x reference)
import jax
import jax.numpy as jnp
from jax.experimental import pallas as pl
from jax.experimental.pallas import tpu as pltpu

HP = 256
G4 = 4 * HP
GW = 768
GOFF = 512
HW = 150
COUT = 128
N_LAYERS = 4
N_CLASSES = 10
N_CHUNKS = 2


def _pack_gates(w):
    i_g = w[..., 0:HW]
    f_g = w[..., HP:HP + HW]
    g_g = w[..., 2 * HP:2 * HP + HW]
    o_g = w[..., 3 * HP:3 * HP + HW]
    sig = jnp.concatenate([i_g, f_g, o_g], axis=-1) * 0.5
    pad_mid = [(0, 0)] * (w.ndim - 1) + [(0, GOFF - 3 * HW)]
    pad_end = [(0, 0)] * (w.ndim - 1) + [(0, GW - GOFF - HW)]
    return jnp.concatenate(
        [jnp.pad(sig, pad_mid), jnp.pad(g_g, pad_end)], axis=-1)


def _cell(z, c_prev):
    ch = z.shape[0]
    s = jnp.tanh(z)
    i_g = 0.5 + 0.5 * s[:, 0:HP]
    f_g = 0.5 + 0.5 * jax.lax.slice(s, (0, HW), (ch, HW + HP))
    o_g = 0.5 + 0.5 * jax.lax.slice(s, (0, 2 * HW), (ch, 2 * HW + HP))
    g_g = s[:, GOFF:GOFF + HP]
    c_new = f_g * c_prev + i_g * g_g
    h_new = o_g * jnp.tanh(c_new)
    return h_new, c_new


def _z3(lhs_x, w_x, lhs_h, w_h, bias, first):
    zs = []
    for n in range(GW // HP):
        z = jnp.dot(lhs_x, w_x[:, n * HP:(n + 1) * HP],
                    preferred_element_type=jnp.float32) + bias[:, n * HP:(n + 1) * HP]
        if not first:
            z += jnp.dot(lhs_h, w_h[:, n * HP:(n + 1) * HP],
                         preferred_element_type=jnp.float32)
        zs.append(z)
    return jnp.concatenate(zs, axis=1)


def _lstm_body(xT_ref, w0_ref, b0_ref, u0_ref, wcat_ref, br_ref, wfc_ref,
               bfc_ref, out_ref, seq_ref):
    T = xT_ref.shape[0]
    bb = out_ref.shape[0]
    C = N_CHUNKS if bb % (8 * N_CHUNKS) == 0 else 1
    ch = bb // C

    w0 = w0_ref[...]
    u0 = u0_ref[...]
    b0 = b0_ref[...]
    cs = [jnp.zeros((ch, HP), jnp.float32)] * C
    hs = [jnp.zeros((ch, HP), jnp.bfloat16)] * C
    for t in range(T):
        for j in range(C):
            z = _z3(xT_ref[t, j * ch:(j + 1) * ch], w0, hs[j], u0, b0, t == 0)
            h, cs[j] = _cell(z, cs[j])
            hs[j] = h.astype(jnp.bfloat16)
            seq_ref[t, j * ch:(j + 1) * ch] = hs[j]

    n_rest = wcat_ref.shape[0]
    for l in range(n_rest):
        wih = wcat_ref[l, :HP]
        whh = wcat_ref[l, HP:]
        b = br_ref[l]
        cs = [jnp.zeros((ch, HP), jnp.float32)] * C
        hs = [jnp.zeros((ch, HP), jnp.bfloat16)] * C
        for t in range(T):
            for j in range(C):
                z = _z3(seq_ref[t, j * ch:(j + 1) * ch], wih, hs[j], whh,
                        b, t == 0)
                h, cs[j] = _cell(z, cs[j])
                hs[j] = h.astype(jnp.bfloat16)
                if l + 1 < n_rest:
                    seq_ref[t, j * ch:(j + 1) * ch] = hs[j]

    for j in range(C):
        out_ref[j * ch:(j + 1) * ch] = (
            jnp.dot(hs[j], wfc_ref[...],
                    preferred_element_type=jnp.float32) + bfc_ref[...])


def kernel(x, w_ih0, b0, u0, wcat, b_rest, wfc, bfc):
    B, T, F = x.shape
    bf = jnp.bfloat16

    if B % 256 == 0 and B >= 512:
        bb = 256
    elif B % 8 == 0 and B > 8:
        bb = 8
    else:
        bb = B
    grid = (B // bb,)

    xT = jnp.transpose(x, (1, 0, 2)).astype(bf)

    w0p = _pack_gates(w_ih0).astype(bf)
    u0p = _pack_gates(u0).astype(bf)
    wcatp = _pack_gates(wcat).astype(bf)
    b0p = _pack_gates(b0)
    brp = _pack_gates(b_rest)

    out = pl.pallas_call(
        _lstm_body,
        out_shape=jax.ShapeDtypeStruct((B, COUT), jnp.float32),
        grid=grid,
        in_specs=[
            pl.BlockSpec((T, bb, F), lambda i: (0, i, 0)),
            pl.BlockSpec((F, GW), lambda i: (0, 0)),
            pl.BlockSpec((1, GW), lambda i: (0, 0)),
            pl.BlockSpec((HP, GW), lambda i: (0, 0)),
            pl.BlockSpec((N_LAYERS - 1, 2 * HP, GW), lambda i: (0, 0, 0)),
            pl.BlockSpec((N_LAYERS - 1, 1, GW), lambda i: (0, 0, 0)),
            pl.BlockSpec((HP, COUT), lambda i: (0, 0)),
            pl.BlockSpec((1, COUT), lambda i: (0, 0)),
        ],
        out_specs=pl.BlockSpec((bb, COUT), lambda i: (i, 0)),
        scratch_shapes=[pltpu.VMEM((T, bb, HP), jnp.bfloat16)],
        compiler_params=pltpu.CompilerParams(
            dimension_semantics=("parallel",),
            vmem_limit_bytes=64 * 1024 * 1024),
    )(xT, w0p, b0p, u0p, wcatp, brp, wfc.astype(bf), bfc)
    return out[:, :N_CLASSES]


# speedup vs baseline: 2.2070x; 2.2070x over previous
import jax
import jax.numpy as jnp
from jax.experimental import pallas as pl
from jax.experimental.pallas import tpu as pltpu

HP = 256
G4 = 4 * HP
GW = 768
GOFF = 512
HW = 150
COUT = 128
N_LAYERS = 4
N_CLASSES = 10
N_CHUNKS = 2


def _pack_gates(w):
    i_g = w[..., 0:HW]
    f_g = w[..., HP:HP + HW]
    g_g = w[..., 2 * HP:2 * HP + HW]
    o_g = w[..., 3 * HP:3 * HP + HW]
    sig = jnp.concatenate([i_g, f_g, o_g], axis=-1) * 0.5
    pad_mid = [(0, 0)] * (w.ndim - 1) + [(0, GOFF - 3 * HW)]
    pad_end = [(0, 0)] * (w.ndim - 1) + [(0, GW - GOFF - HW)]
    return jnp.concatenate(
        [jnp.pad(sig, pad_mid), jnp.pad(g_g, pad_end)], axis=-1)


def _cell(z, c_prev):
    s = jnp.tanh(z)
    i_g = 0.5 + 0.5 * s[:, 0:HP]
    f_g = 0.5 + 0.5 * jnp.roll(s[:, 128:384], -(HW - 128), axis=1)
    o_g = 0.5 + 0.5 * jnp.roll(s[:, 256:512], -(2 * HW - 256), axis=1)
    g_g = s[:, GOFF:GOFF + HP]
    c_new = f_g * c_prev + i_g * g_g
    h_new = o_g * jnp.tanh(c_new)
    return h_new, c_new


def _z3(lhs_x, w_x, lhs_h, w_h, bias, first):
    zs = []
    for n in range(GW // HP):
        z = jnp.dot(lhs_x, w_x[:, n * HP:(n + 1) * HP],
                    preferred_element_type=jnp.float32) + bias[:, n * HP:(n + 1) * HP]
        if not first:
            z += jnp.dot(lhs_h, w_h[:, n * HP:(n + 1) * HP],
                         preferred_element_type=jnp.float32)
        zs.append(z)
    return jnp.concatenate(zs, axis=1)


def _lstm_body(xT_ref, w0_ref, b0_ref, u0_ref, wcat_ref, br_ref, wfc_ref,
               bfc_ref, out_ref, seq_ref):
    T = xT_ref.shape[0]
    bb = out_ref.shape[0]
    C = N_CHUNKS if bb % (8 * N_CHUNKS) == 0 else 1
    ch = bb // C

    w0 = w0_ref[...]
    u0 = u0_ref[...]
    b0 = b0_ref[...]
    cs = [jnp.zeros((ch, HP), jnp.float32)] * C
    hs = [jnp.zeros((ch, HP), jnp.bfloat16)] * C
    for t in range(T):
        for j in range(C):
            z = _z3(xT_ref[t, j * ch:(j + 1) * ch], w0, hs[j], u0, b0, t == 0)
            h, cs[j] = _cell(z, cs[j])
            hs[j] = h.astype(jnp.bfloat16)
            seq_ref[t, j * ch:(j + 1) * ch] = hs[j]

    n_rest = wcat_ref.shape[0]
    for l in range(n_rest):
        wih = wcat_ref[l, :HP]
        whh = wcat_ref[l, HP:]
        b = br_ref[l]
        cs = [jnp.zeros((ch, HP), jnp.float32)] * C
        hs = [jnp.zeros((ch, HP), jnp.bfloat16)] * C
        for t in range(T):
            for j in range(C):
                z = _z3(seq_ref[t, j * ch:(j + 1) * ch], wih, hs[j], whh,
                        b, t == 0)
                h, cs[j] = _cell(z, cs[j])
                hs[j] = h.astype(jnp.bfloat16)
                if l + 1 < n_rest:
                    seq_ref[t, j * ch:(j + 1) * ch] = hs[j]

    for j in range(C):
        out_ref[j * ch:(j + 1) * ch] = (
            jnp.dot(hs[j], wfc_ref[...],
                    preferred_element_type=jnp.float32) + bfc_ref[...])


def kernel(x, w_ih0, b0, u0, wcat, b_rest, wfc, bfc):
    B, T, F = x.shape
    bf = jnp.bfloat16

    if B % 256 == 0 and B >= 512:
        bb = 256
    elif B % 8 == 0 and B > 8:
        bb = 8
    else:
        bb = B
    grid = (B // bb,)

    xT = jnp.transpose(x, (1, 0, 2)).astype(bf)

    w0p = _pack_gates(w_ih0).astype(bf)
    u0p = _pack_gates(u0).astype(bf)
    wcatp = _pack_gates(wcat).astype(bf)
    b0p = _pack_gates(b0)
    brp = _pack_gates(b_rest)

    out = pl.pallas_call(
        _lstm_body,
        out_shape=jax.ShapeDtypeStruct((B, COUT), jnp.float32),
        grid=grid,
        in_specs=[
            pl.BlockSpec((T, bb, F), lambda i: (0, i, 0)),
            pl.BlockSpec((F, GW), lambda i: (0, 0)),
            pl.BlockSpec((1, GW), lambda i: (0, 0)),
            pl.BlockSpec((HP, GW), lambda i: (0, 0)),
            pl.BlockSpec((N_LAYERS - 1, 2 * HP, GW), lambda i: (0, 0, 0)),
            pl.BlockSpec((N_LAYERS - 1, 1, GW), lambda i: (0, 0, 0)),
            pl.BlockSpec((HP, COUT), lambda i: (0, 0)),
            pl.BlockSpec((1, COUT), lambda i: (0, 0)),
        ],
        out_specs=pl.BlockSpec((bb, COUT), lambda i: (i, 0)),
        scratch_shapes=[pltpu.VMEM((T, bb, HP), jnp.bfloat16)],
        compiler_params=pltpu.CompilerParams(
            dimension_semantics=("parallel",),
            vmem_limit_bytes=64 * 1024 * 1024),
    )(xT, w0p, b0p, u0p, wcatp, brp, wfc.astype(bf), bfc)
    return out[:, :N_CLASSES]
